# baseline (speedup 1.0000x reference)
import functools

import jax
import jax.numpy as jnp
from jax import lax
from jax.experimental import pallas as pl
from jax.experimental.pallas import tpu as pltpu
from jax.experimental.pallas import tpu_sc as plsc

B = 16384
N_FIELDS = 26
FIELD_DIM = 1000
TOTAL = N_FIELDS * FIELD_DIM
NPACK = 9

NUM_CORES = 2
NUM_SUBCORES = 16
LANES = 16
NW = NUM_CORES * NUM_SUBCORES
BPW = B // NW
NCHUNK = BPW // LANES

_mesh = plsc.VectorSubcoreMesh(core_axis_name="c", subcore_axis_name="s")


def _tree_sum(vs):
    while len(vs) > 1:
        pairs = [vs[i] + vs[i + 1] for i in range(0, len(vs) - 1, 2)]
        if len(vs) % 2:
            pairs.append(vs[-1])
        vs = pairs
    return vs[0]


@functools.partial(
    pl.kernel,
    out_type=jax.ShapeDtypeStruct((B,), jnp.float32),
    mesh=_mesh,
    scratch_types=[
        pltpu.VMEM((TOTAL,), jnp.float32),
        pltpu.VMEM((NPACK, BPW), jnp.int32),
        pltpu.VMEM((BPW,), jnp.float32),
        pltpu.VMEM((1,), jnp.float32),
        pltpu.SemaphoreType.DMA,
        pltpu.SemaphoreType.DMA,
    ],
    compiler_params=pltpu.CompilerParams(needs_layout_passes=False),
)
def _features_linear(xp_hbm, w_hbm, b_hbm, out_hbm,
                     w_v, xp_v, out_v, b_v, sem_w, sem_x):
    wid = lax.axis_index("s") * NUM_CORES + lax.axis_index("c")
    base = wid * BPW
    cp_w = pltpu.async_copy(w_hbm, w_v, sem_w)
    cp_x = pltpu.async_copy(xp_hbm.at[wid], xp_v, sem_x)
    pltpu.sync_copy(b_hbm, b_v)
    cp_x.wait()
    cp_w.wait()
    bias = plsc.load_gather(b_v, [jnp.zeros((LANES,), jnp.int32)])

    @plsc.parallel_loop(0, NCHUNK, unroll=2)
    def chunk(c):
        terms = []
        for j in range(NPACK):
            word = xp_v[j, pl.ds(c * LANES, LANES)]
            for s in range(3):
                f = 3 * j + s
                if f >= N_FIELDS:
                    break
                idx = ((word >> (10 * s)) & 1023) + (f * FIELD_DIM)
                terms.append(plsc.load_gather(w_v, [idx]))
        out_v[pl.ds(c * LANES, LANES)] = _tree_sum(terms) + bias

    pltpu.sync_copy(out_v, out_hbm.at[pl.ds(base, BPW)])


def kernel(x, offsets, weight, bias):
    del offsets
    xi = x.astype(jnp.int32)
    p0 = xi[:, 0::3]
    p1 = xi[:, 1::3]
    p2 = jnp.pad(xi[:, 2::3], ((0, 0), (0, 1)))
    packed = p0 | (p1 << 10) | (p2 << 20)
    xp = packed.reshape(NW, BPW, NPACK).transpose(0, 2, 1)
    out = _features_linear(xp, weight.reshape(TOTAL), bias)
    return out[:, None]


# speedup vs baseline: 1.2347x; 1.2347x over previous
import functools

import jax
import jax.numpy as jnp
from jax import lax
from jax.experimental import pallas as pl
from jax.experimental.pallas import tpu as pltpu
from jax.experimental.pallas import tpu_sc as plsc

B = 16384
N_FIELDS = 26
FIELD_DIM = 1000
TOTAL = N_FIELDS * FIELD_DIM

NUM_CORES = 2
NUM_SUBCORES = 16
LANES = 16
NW = NUM_CORES * NUM_SUBCORES
BPW = B // NW
NCHUNK = BPW // LANES

_mesh = plsc.VectorSubcoreMesh(core_axis_name="c", subcore_axis_name="s")


def _tree_sum(vs):
    while len(vs) > 1:
        pairs = [vs[i] + vs[i + 1] for i in range(0, len(vs) - 1, 2)]
        if len(vs) % 2:
            pairs.append(vs[-1])
        vs = pairs
    return vs[0]


@functools.partial(
    pl.kernel,
    out_type=jax.ShapeDtypeStruct((B,), jnp.float32),
    mesh=_mesh,
    scratch_types=[
        pltpu.VMEM((TOTAL,), jnp.float32),
        pltpu.VMEM((N_FIELDS, BPW), jnp.int32),
        pltpu.VMEM((BPW,), jnp.float32),
        pltpu.VMEM((1,), jnp.float32),
        pltpu.SemaphoreType.DMA,
        pltpu.SemaphoreType.DMA,
    ],
    compiler_params=pltpu.CompilerParams(needs_layout_passes=False),
)
def _features_linear(xt_hbm, w_hbm, b_hbm, out_hbm,
                     w_v, xt_v, out_v, b_v, sem_w, sem_x):
    wid = lax.axis_index("s") * NUM_CORES + lax.axis_index("c")
    base = wid * BPW
    half = TOTAL // 2
    lo_src, lo_dst = w_hbm.at[pl.ds(0, half)], w_v.at[pl.ds(0, half)]
    hi_src, hi_dst = w_hbm.at[pl.ds(half, half)], w_v.at[pl.ds(half, half)]

    @pl.when(wid % 2 == 0)
    def _():
        pltpu.async_copy(lo_src, lo_dst, sem_w)
        pltpu.async_copy(hi_src, hi_dst, sem_w)

    @pl.when(wid % 2 == 1)
    def _():
        pltpu.async_copy(hi_src, hi_dst, sem_w)
        pltpu.async_copy(lo_src, lo_dst, sem_w)

    cp_x = pltpu.async_copy(xt_hbm.at[wid], xt_v, sem_x)
    pltpu.sync_copy(b_hbm, b_v)
    cp_x.wait()
    pltpu.make_async_copy(lo_src, lo_dst, sem_w).wait()
    pltpu.make_async_copy(hi_src, hi_dst, sem_w).wait()
    bias = plsc.load_gather(b_v, [jnp.zeros((LANES,), jnp.int32)])

    @plsc.parallel_loop(0, NCHUNK, unroll=2)
    def chunk(c):
        terms = []
        for f in range(N_FIELDS):
            idx = xt_v[f, pl.ds(c * LANES, LANES)] + (f * FIELD_DIM)
            terms.append(plsc.load_gather(w_v, [idx]))
        out_v[pl.ds(c * LANES, LANES)] = _tree_sum(terms) + bias

    pltpu.sync_copy(out_v, out_hbm.at[pl.ds(base, BPW)])


def kernel(x, offsets, weight, bias):
    del offsets
    xt = x.astype(jnp.int32).reshape(NW, BPW, N_FIELDS).transpose(0, 2, 1)
    out = _features_linear(xt, weight.reshape(TOTAL), bias)
    return out[:, None]


# speedup vs baseline: 1.2380x; 1.0027x over previous
import functools

import jax
import jax.numpy as jnp
from jax import lax
from jax.experimental import pallas as pl
from jax.experimental.pallas import tpu as pltpu
from jax.experimental.pallas import tpu_sc as plsc

B = 16384
N_FIELDS = 26
FIELD_DIM = 1000
TOTAL = N_FIELDS * FIELD_DIM

NUM_CORES = 2
NUM_SUBCORES = 16
LANES = 16
NW = NUM_CORES * NUM_SUBCORES
BPW = B // NW
NCHUNK = BPW // LANES

_mesh = plsc.VectorSubcoreMesh(core_axis_name="c", subcore_axis_name="s")


def _tree_sum(vs):
    while len(vs) > 1:
        pairs = [vs[i] + vs[i + 1] for i in range(0, len(vs) - 1, 2)]
        if len(vs) % 2:
            pairs.append(vs[-1])
        vs = pairs
    return vs[0]


@functools.partial(
    pl.kernel,
    out_type=jax.ShapeDtypeStruct((B,), jnp.float32),
    mesh=_mesh,
    scratch_types=[
        pltpu.VMEM((TOTAL,), jnp.float32),
        pltpu.VMEM((N_FIELDS, BPW), jnp.int32),
        pltpu.VMEM((BPW,), jnp.float32),
        pltpu.VMEM((1,), jnp.float32),
        pltpu.SemaphoreType.DMA,
        pltpu.SemaphoreType.DMA,
    ],
    compiler_params=pltpu.CompilerParams(needs_layout_passes=False),
)
def _features_linear(xt_hbm, w_hbm, b_hbm, out_hbm,
                     w_v, xt_v, out_v, b_v, sem_w, sem_x):
    wid = lax.axis_index("s") * NUM_CORES + lax.axis_index("c")
    base = wid * BPW
    bounds = (0, 6504, 13000, 19504, TOTAL)
    segs = [(w_hbm.at[pl.ds(bounds[q], bounds[q + 1] - bounds[q])],
             w_v.at[pl.ds(bounds[q], bounds[q + 1] - bounds[q])])
            for q in range(4)]
    for r in range(4):
        @pl.when(wid % 4 == r)
        def _(r=r):
            for k in range(4):
                src, dst = segs[(r + k) % 4]
                pltpu.async_copy(src, dst, sem_w)

    cp_x = pltpu.async_copy(xt_hbm.at[wid], xt_v, sem_x)
    pltpu.sync_copy(b_hbm, b_v)
    cp_x.wait()
    for src, dst in segs:
        pltpu.make_async_copy(src, dst, sem_w).wait()
    bias = plsc.load_gather(b_v, [jnp.zeros((LANES,), jnp.int32)])

    @plsc.parallel_loop(0, NCHUNK, unroll=2)
    def chunk(c):
        terms = []
        for f in range(N_FIELDS):
            idx = xt_v[f, pl.ds(c * LANES, LANES)] + (f * FIELD_DIM)
            terms.append(plsc.load_gather(w_v, [idx]))
        out_v[pl.ds(c * LANES, LANES)] = _tree_sum(terms) + bias

    pltpu.sync_copy(out_v, out_hbm.at[pl.ds(base, BPW)])


def kernel(x, offsets, weight, bias):
    del offsets
    xt = x.astype(jnp.int32).reshape(NW, BPW, N_FIELDS).transpose(0, 2, 1)
    out = _features_linear(xt, weight.reshape(TOTAL), bias)
    return out[:, None]
